# baseline (speedup 1.0000x reference)
import functools

import jax
import jax.numpy as jnp
from jax import lax
from jax.experimental import pallas as pl
from jax.experimental.pallas import tpu as pltpu
from jax.experimental.pallas import tpu_sc as plsc

N = 10000
D = 128
INNER = 20
ENC = 16

NC = 2
NS = 16
CHUNK = 128

NPAD = 10112
NBUF = 4
RPT = NPAD // NS
R1 = 24
R2 = 16

_HI = lax.Precision.HIGHEST


def _dot(a, b):
    return lax.dot_general(a, b, (((1,), (0,)), ((), ())), precision=_HI)


def _make_sc_scatter(R, k0, k1, klast):
    mesh = plsc.VectorSubcoreMesh(core_axis_name="c", subcore_axis_name="s")
    kmax = max(k0, k1, klast)

    def body(edges, table, zeros, out, src_v, dst_v, r0, r1, r2, r3,
             acc, g0, g1, g2, g3):
        rows = (r0, r1, r2, r3)
        gsem = (g0, g1, g2, g3)
        c = lax.axis_index("c")
        s = lax.axis_index("s")
        pltpu.sync_copy(zeros.at[pl.ds(s * RPT, RPT)],
                        acc.at[pl.ds(s * RPT, RPT)])
        k = lax.select(c == 0, lax.select(s == NS - 1, klast, k0), k1)
        lo = lax.select(c == 0, s * k0,
                        (NS - 1) * k0 + klast + s * k1)

        @pl.when((c == 0) & (s < NS - 1))
        def _():
            pltpu.sync_copy(edges.at[0, pl.ds(lo, k0)],
                            src_v.at[pl.ds(0, k0)])
            pltpu.sync_copy(edges.at[1, pl.ds(lo, k0)],
                            dst_v.at[pl.ds(0, k0)])

        @pl.when((c == 0) & (s == NS - 1))
        def _():
            pltpu.sync_copy(edges.at[0, pl.ds(lo, klast)],
                            src_v.at[pl.ds(0, klast)])
            pltpu.sync_copy(edges.at[1, pl.ds(lo, klast)],
                            dst_v.at[pl.ds(0, klast)])

        @pl.when(c == 1)
        def _():
            pltpu.sync_copy(edges.at[0, pl.ds(lo, k1)],
                            src_v.at[pl.ds(0, k1)])
            pltpu.sync_copy(edges.at[1, pl.ds(lo, k1)],
                            dst_v.at[pl.ds(0, k1)])

        plsc.subcore_barrier()

        for b in range(NBUF):
            pltpu.async_copy(table.at[src_v.at[b]], rows[b], gsem[b])

        def step(i, carry):
            base = i * NBUF
            for b in range(NBUF):
                pltpu.make_async_copy(table.at[src_v.at[0]], rows[b],
                                      gsem[b]).wait()
                pltpu.sync_copy(rows[b], acc.at[dst_v.at[base + b]],
                                add=True)

                @pl.when(base + NBUF + b < k)
                def _():
                    pltpu.async_copy(table.at[src_v.at[base + NBUF + b]],
                                     rows[b], gsem[b])
            return carry

        lax.fori_loop(0, k // NBUF, step, 0, unroll=False)
        plsc.subcore_barrier()
        pltpu.sync_copy(acc.at[pl.ds(s * RPT, RPT)],
                        out.at[c, pl.ds(s * RPT, RPT)])

    return pl.kernel(
        body,
        mesh=mesh,
        compiler_params=pltpu.CompilerParams(use_tc_tiling_on_sc=False),
        out_type=jax.ShapeDtypeStruct((NC, NPAD, R), jnp.float32),
        scratch_types=(
            [pltpu.VMEM((kmax, CHUNK), jnp.int32)] * 2
            + [pltpu.VMEM((CHUNK, R), jnp.float32)] * NBUF
            + [pltpu.VMEM_SHARED((NPAD, R), jnp.float32)]
            + [pltpu.SemaphoreType.DMA] * NBUF
        ),
    )


def _split_chunks(ktot):
    k1 = max(NBUF, int(round(ktot / (NS * 5.0) / NBUF)) * NBUF)
    k0 = (ktot - NS * k1) // (NS - 1) // NBUF * NBUF
    klast = ktot - (NS - 1) * k0 - NS * k1
    assert k0 >= NBUF and klast >= NBUF, (k0, k1, klast)
    return k0, k1, klast


def _table1_body(x_ref, w1_ref, y_ref):
    y = _dot(x_ref[...], w1_ref[D:, :])
    ones = jnp.ones((N, 1), jnp.float32)
    pad = jnp.zeros((N, R1 - INNER - 1), jnp.float32)
    y_ref[...] = jnp.concatenate([y, ones, pad], axis=1)


NBLK = 1
BLK = N // NBLK


def _mid1_body(u_ref, x_ref, w1_ref, b1_ref, h_ref, stats_ref, acc):
    i = pl.program_id(0)

    @pl.when(i == 0)
    def _():
        acc[...] = jnp.zeros_like(acc)

    u = u_ref[0] + u_ref[1]
    seg = u[:, :INNER]
    deg = u[:, INNER:INNER + 1]
    wd1 = w1_ref[:D, :] - w1_ref[D:, :]
    h = deg * (_dot(x_ref[...], wd1) + b1_ref[...]) + seg
    h_ref[...] = h
    acc[...] += jnp.concatenate(
        [jnp.sum(h, axis=0, keepdims=True),
         jnp.sum(h * h, axis=0, keepdims=True)], axis=0)
    stats_ref[...] = acc[...]


def _mid2_body(h_ref, stats_ref, u_ref, g_ref, be_ref, w2_ref, b2_ref,
               m_ref, od_ref):
    h = h_ref[...]
    deg = u_ref[0, :, INNER:INNER + 1] + u_ref[1, :, INNER:INNER + 1]
    mean = stats_ref[0:1, :] * (1.0 / N)
    var = stats_ref[1:2, :] * (1.0 / N) - mean * mean
    hp = jnp.maximum((h - mean) * lax.rsqrt(var + 1e-5) * g_ref[...]
                     + be_ref[...], 0.0)
    w2b = w2_ref[INNER:, :]
    m_ref[...] = _dot(hp, w2b)
    od_ref[...] = deg * (_dot(hp, w2_ref[:INNER, :] - w2b) + b2_ref[...])


def _fin_body(od_ref, p_ref, out_ref):
    out_ref[...] = od_ref[...] + p_ref[0, :N] + p_ref[1, :N]


_sc_scatter_cache = {}


def _sc_scatter(R, k0, k1, klast):
    key = (R, k0, k1, klast)
    if key not in _sc_scatter_cache:
        _sc_scatter_cache[key] = _make_sc_scatter(R, k0, k1, klast)
    return _sc_scatter_cache[key]


def kernel(x, edge_index, W1, b1, gamma, beta, W2, b2):
    f32 = jnp.float32
    e = edge_index.shape[1]
    ktot = -(-e // CHUNK)
    ktot = -(-ktot // NBUF) * NBUF
    epad = ktot * CHUNK
    if epad != e:
        edge_index = jnp.concatenate(
            [edge_index,
             jnp.stack([jnp.zeros((epad - e,), jnp.int32),
                        jnp.full((epad - e,), N, jnp.int32)])], axis=1)
    edges = edge_index.reshape(2, ktot, CHUNK)
    k0, k1, klast = _split_chunks(ktot)
    zeros1 = jnp.zeros((NPAD, R1), f32)
    zeros2 = jnp.zeros((NPAD, R2), f32)

    y_ext = pl.pallas_call(
        _table1_body,
        out_shape=jax.ShapeDtypeStruct((N, R1), f32),
    )(x, W1)

    u = _sc_scatter(R1, k0, k1, klast)(edges, y_ext, zeros1)

    h, stats = pl.pallas_call(
        _mid1_body,
        grid=(NBLK,),
        in_specs=[
            pl.BlockSpec((2, BLK, R1), lambda i: (0, i, 0)),
            pl.BlockSpec((BLK, D), lambda i: (i, 0)),
            pl.BlockSpec((2 * D, INNER), lambda i: (0, 0)),
            pl.BlockSpec((1, INNER), lambda i: (0, 0)),
        ],
        out_specs=[
            pl.BlockSpec((BLK, INNER), lambda i: (i, 0)),
            pl.BlockSpec((2, INNER), lambda i: (0, 0)),
        ],
        out_shape=[jax.ShapeDtypeStruct((N, INNER), f32),
                   jax.ShapeDtypeStruct((2, INNER), f32)],
        scratch_shapes=[pltpu.VMEM((2, INNER), f32)],
    )(u, x, W1, b1.reshape(1, INNER))

    m, od = pl.pallas_call(
        _mid2_body,
        grid=(NBLK,),
        in_specs=[
            pl.BlockSpec((BLK, INNER), lambda i: (i, 0)),
            pl.BlockSpec((2, INNER), lambda i: (0, 0)),
            pl.BlockSpec((2, BLK, R1), lambda i: (0, i, 0)),
            pl.BlockSpec((1, INNER), lambda i: (0, 0)),
            pl.BlockSpec((1, INNER), lambda i: (0, 0)),
            pl.BlockSpec((2 * INNER, ENC), lambda i: (0, 0)),
            pl.BlockSpec((1, ENC), lambda i: (0, 0)),
        ],
        out_specs=[
            pl.BlockSpec((BLK, ENC), lambda i: (i, 0)),
            pl.BlockSpec((BLK, ENC), lambda i: (i, 0)),
        ],
        out_shape=[jax.ShapeDtypeStruct((N, ENC), f32),
                   jax.ShapeDtypeStruct((N, ENC), f32)],
    )(h, stats, u, gamma.reshape(1, INNER), beta.reshape(1, INNER),
      W2, b2.reshape(1, ENC))

    p = _sc_scatter(R2, k0, k1, klast)(edges, m, zeros2)

    out = pl.pallas_call(
        _fin_body,
        out_shape=jax.ShapeDtypeStruct((N, ENC), f32),
    )(od, p)
    return out


# speedup vs baseline: 1.0647x; 1.0647x over previous
import functools

import jax
import jax.numpy as jnp
from jax import lax
from jax.experimental import pallas as pl
from jax.experimental.pallas import tpu as pltpu
from jax.experimental.pallas import tpu_sc as plsc

N = 10000
D = 128
INNER = 20
ENC = 16

NC = 2
NS = 16
CHUNK = 128

NPAD = 10112
NBUF = 4
RPT = NPAD // NS
R1 = 24
R2 = 16

_HI = lax.Precision.HIGHEST


def _dot(a, b):
    return lax.dot_general(a, b, (((1,), (0,)), ((), ())), precision=_HI)


def _make_sc_scatter(R, k0, k1, klast):
    mesh = plsc.VectorSubcoreMesh(core_axis_name="c", subcore_axis_name="s")
    kmax = max(k0, k1, klast)

    def body(edges, table, zeros, out, src_v, dst_v, r0, r1, r2, r3,
             acc, g0, g1, g2, g3):
        rows = (r0, r1, r2, r3)
        gsem = (g0, g1, g2, g3)
        c = lax.axis_index("c")
        s = lax.axis_index("s")
        pltpu.sync_copy(zeros.at[pl.ds(s * RPT, RPT)],
                        acc.at[pl.ds(s * RPT, RPT)])
        k = lax.select(c == 0, k0, lax.select(s == NS - 1, klast, k1))
        lo = lax.select(c == 0, s * k0, NS * k0 + s * k1)

        @pl.when(c == 0)
        def _():
            pltpu.sync_copy(edges.at[0, pl.ds(lo, k0)],
                            src_v.at[pl.ds(0, k0)])
            pltpu.sync_copy(edges.at[1, pl.ds(lo, k0)],
                            dst_v.at[pl.ds(0, k0)])

        @pl.when((c == 1) & (s < NS - 1))
        def _():
            pltpu.sync_copy(edges.at[0, pl.ds(lo, k1)],
                            src_v.at[pl.ds(0, k1)])
            pltpu.sync_copy(edges.at[1, pl.ds(lo, k1)],
                            dst_v.at[pl.ds(0, k1)])

        @pl.when((c == 1) & (s == NS - 1))
        def _():
            pltpu.sync_copy(edges.at[0, pl.ds(lo, klast)],
                            src_v.at[pl.ds(0, klast)])
            pltpu.sync_copy(edges.at[1, pl.ds(lo, klast)],
                            dst_v.at[pl.ds(0, klast)])

        plsc.subcore_barrier()

        for b in range(NBUF):
            pltpu.async_copy(table.at[src_v.at[b]], rows[b], gsem[b])

        def step(i, carry):
            base = i * NBUF
            for b in range(NBUF):
                pltpu.make_async_copy(table.at[src_v.at[0]], rows[b],
                                      gsem[b]).wait()
                pltpu.sync_copy(rows[b], acc.at[dst_v.at[base + b]],
                                add=True)

                @pl.when(base + NBUF + b < k)
                def _():
                    pltpu.async_copy(table.at[src_v.at[base + NBUF + b]],
                                     rows[b], gsem[b])
            return carry

        lax.fori_loop(0, k // NBUF, step, 0, unroll=False)
        plsc.subcore_barrier()
        pltpu.sync_copy(acc.at[pl.ds(s * RPT, RPT)],
                        out.at[c, pl.ds(s * RPT, RPT)])

    return pl.kernel(
        body,
        mesh=mesh,
        compiler_params=pltpu.CompilerParams(use_tc_tiling_on_sc=False),
        out_type=jax.ShapeDtypeStruct((NC, NPAD, R), jnp.float32),
        scratch_types=(
            [pltpu.VMEM((kmax, CHUNK), jnp.int32)] * 2
            + [pltpu.VMEM((CHUNK, R), jnp.float32)] * NBUF
            + [pltpu.VMEM_SHARED((NPAD, R), jnp.float32)]
            + [pltpu.SemaphoreType.DMA] * NBUF
        ),
    )


def _split_chunks(ktot):
    k0 = int(round(ktot * 0.69 / NS / NBUF)) * NBUF
    rem = ktot - NS * k0
    k1 = max(NBUF, rem // (NS - 1) // NBUF * NBUF)
    klast = rem - (NS - 1) * k1
    assert klast >= NBUF and k0 >= NBUF, (k0, k1, klast)
    return k0, k1, klast


def _table1_body(x_ref, w1_ref, y_ref):
    y = _dot(x_ref[...], w1_ref[D:, :])
    ones = jnp.ones((N, 1), jnp.float32)
    pad = jnp.zeros((N, R1 - INNER - 1), jnp.float32)
    y_ref[...] = jnp.concatenate([y, ones, pad], axis=1)


NBLK = 1
BLK = N // NBLK


def _mid1_body(u_ref, x_ref, w1_ref, b1_ref, h_ref, stats_ref, acc):
    i = pl.program_id(0)

    @pl.when(i == 0)
    def _():
        acc[...] = jnp.zeros_like(acc)

    u = u_ref[0] + u_ref[1]
    seg = u[:, :INNER]
    deg = u[:, INNER:INNER + 1]
    wd1 = w1_ref[:D, :] - w1_ref[D:, :]
    h = deg * (_dot(x_ref[...], wd1) + b1_ref[...]) + seg
    h_ref[...] = h
    acc[...] += jnp.concatenate(
        [jnp.sum(h, axis=0, keepdims=True),
         jnp.sum(h * h, axis=0, keepdims=True)], axis=0)
    stats_ref[...] = acc[...]


def _mid2_body(h_ref, stats_ref, u_ref, g_ref, be_ref, w2_ref, b2_ref,
               m_ref, od_ref):
    h = h_ref[...]
    deg = u_ref[0, :, INNER:INNER + 1] + u_ref[1, :, INNER:INNER + 1]
    mean = stats_ref[0:1, :] * (1.0 / N)
    var = stats_ref[1:2, :] * (1.0 / N) - mean * mean
    hp = jnp.maximum((h - mean) * lax.rsqrt(var + 1e-5) * g_ref[...]
                     + be_ref[...], 0.0)
    w2b = w2_ref[INNER:, :]
    m_ref[...] = _dot(hp, w2b)
    od_ref[...] = deg * (_dot(hp, w2_ref[:INNER, :] - w2b) + b2_ref[...])


def _fin_body(od_ref, p_ref, out_ref):
    out_ref[...] = od_ref[...] + p_ref[0, :N] + p_ref[1, :N]


_sc_scatter_cache = {}


def _sc_scatter(R, k0, k1, klast):
    key = (R, k0, k1, klast)
    if key not in _sc_scatter_cache:
        _sc_scatter_cache[key] = _make_sc_scatter(R, k0, k1, klast)
    return _sc_scatter_cache[key]


def kernel(x, edge_index, W1, b1, gamma, beta, W2, b2):
    f32 = jnp.float32
    e = edge_index.shape[1]
    ktot = -(-e // CHUNK)
    ktot = -(-ktot // NBUF) * NBUF
    epad = ktot * CHUNK
    if epad != e:
        edge_index = jnp.concatenate(
            [edge_index,
             jnp.stack([jnp.zeros((epad - e,), jnp.int32),
                        jnp.full((epad - e,), N, jnp.int32)])], axis=1)
    edges = edge_index.reshape(2, ktot, CHUNK)
    k0, k1, klast = _split_chunks(ktot)
    zeros1 = jnp.zeros((NPAD, R1), f32)
    zeros2 = jnp.zeros((NPAD, R2), f32)

    y_ext = pl.pallas_call(
        _table1_body,
        out_shape=jax.ShapeDtypeStruct((N, R1), f32),
    )(x, W1)

    u = _sc_scatter(R1, k0, k1, klast)(edges, y_ext, zeros1)

    h, stats = pl.pallas_call(
        _mid1_body,
        grid=(NBLK,),
        in_specs=[
            pl.BlockSpec((2, BLK, R1), lambda i: (0, i, 0)),
            pl.BlockSpec((BLK, D), lambda i: (i, 0)),
            pl.BlockSpec((2 * D, INNER), lambda i: (0, 0)),
            pl.BlockSpec((1, INNER), lambda i: (0, 0)),
        ],
        out_specs=[
            pl.BlockSpec((BLK, INNER), lambda i: (i, 0)),
            pl.BlockSpec((2, INNER), lambda i: (0, 0)),
        ],
        out_shape=[jax.ShapeDtypeStruct((N, INNER), f32),
                   jax.ShapeDtypeStruct((2, INNER), f32)],
        scratch_shapes=[pltpu.VMEM((2, INNER), f32)],
    )(u, x, W1, b1.reshape(1, INNER))

    m, od = pl.pallas_call(
        _mid2_body,
        grid=(NBLK,),
        in_specs=[
            pl.BlockSpec((BLK, INNER), lambda i: (i, 0)),
            pl.BlockSpec((2, INNER), lambda i: (0, 0)),
            pl.BlockSpec((2, BLK, R1), lambda i: (0, i, 0)),
            pl.BlockSpec((1, INNER), lambda i: (0, 0)),
            pl.BlockSpec((1, INNER), lambda i: (0, 0)),
            pl.BlockSpec((2 * INNER, ENC), lambda i: (0, 0)),
            pl.BlockSpec((1, ENC), lambda i: (0, 0)),
        ],
        out_specs=[
            pl.BlockSpec((BLK, ENC), lambda i: (i, 0)),
            pl.BlockSpec((BLK, ENC), lambda i: (i, 0)),
        ],
        out_shape=[jax.ShapeDtypeStruct((N, ENC), f32),
                   jax.ShapeDtypeStruct((N, ENC), f32)],
    )(h, stats, u, gamma.reshape(1, INNER), beta.reshape(1, INNER),
      W2, b2.reshape(1, ENC))

    p = _sc_scatter(R2, k0, k1, klast)(edges, m, zeros2)

    out = pl.pallas_call(
        _fin_body,
        out_shape=jax.ShapeDtypeStruct((N, ENC), f32),
    )(od, p)
    return out
